# Initial kernel scaffold; baseline (speedup 1.0000x reference)
import jax
import jax.numpy as jnp
from jax.experimental import pallas as pl


def kernel(x, edge_index, batch, W1, b1, Wg, Wih, Whh, bih, bhh, Wf, bf):
    raise NotImplementedError("write your pallas kernel here")


# speedup vs baseline: 4.4840x; 4.4840x over previous
import functools

import jax
import jax.numpy as jnp
from jax import lax
from jax.experimental import pallas as pl
from jax.experimental.pallas import tpu as pltpu
from jax.experimental.pallas import tpu_sc as plsc

N = 10000
E = 320000
D = 128
H = 32
C = 7
L = 32

_NPAD = 12288
_ZR = _NPAD // 16
_LOCAL = 2048

_CUTS_E = [0]
for _sc in (0, 160000):
    _CUTS_E += [_sc + 10368 + 9984 * k for k in range(15)] + [_sc + 160000]
_CUTS_G = [0] + [20736 * k for k in range(1, 13)] \
    + [248832 + 20352 * k for k in range(1, 4)] + [330000]

_SH_E = 81
_SH_G = 162


def _make_sorted_scatter(W, nsh, shmax, scaled):
    mesh = plsc.VectorSubcoreMesh(core_axis_name="c", subcore_axis_name="s")

    out_type = [
        jax.ShapeDtypeStruct((2, N, W), jnp.float32),
        jax.ShapeDtypeStruct((nsh, 2, W), jnp.float32),
    ]
    scratch = [
        pltpu.VMEM((128,), jnp.int32),
        pltpu.VMEM((128,), jnp.int32),
        pltpu.VMEM((128, W), jnp.float32),
        pltpu.VMEM((_LOCAL, W), jnp.float32),
        pltpu.VMEM((8, W), jnp.float32),
        pltpu.VMEM((nsh, 16), jnp.int32),
        pltpu.VMEM((_LOCAL // 128, 128), jnp.int32),
        pltpu.VMEM_SHARED((_NPAD, W), jnp.float32),
    ]
    if scaled:
        scratch += [pltpu.VMEM((N,), jnp.float32),
                    pltpu.VMEM((128,), jnp.float32)]

    @functools.partial(
        pl.kernel,
        out_type=out_type,
        mesh=mesh,
        scratch_types=scratch,
        compiler_params=pltpu.CompilerParams(use_tc_tiling_on_sc=False, needs_layout_passes=False),
    )
    def body(table, ssrc3, sdst3, meta, zero, *rest):
        if scaled:
            (dinv, out_acc, out_side, ssrc_c, sdst_c, rows_v, acc_v, side_v,
             meta_v, idx_v, acc_sh, dinv_v, norm_v) = rest
        else:
            (out_acc, out_side, ssrc_c, sdst_c, rows_v, acc_v, side_v,
             meta_v, idx_v, acc_sh) = rest
        cid = lax.axis_index("c")
        sid = lax.axis_index("s")
        wid = sid * 2 + cid
        pltpu.sync_copy(zero.at[pl.ds(0, _ZR)], acc_sh.at[pl.ds(sid * _ZR, _ZR)])

        @pl.when(wid < nsh)
        def _work():
            pltpu.sync_copy(zero, acc_v)
            pltpu.sync_copy(meta, meta_v)
            if scaled:
                pltpu.sync_copy(dinv, dinv_v)
            mrow = meta_v[wid]
            ln = mrow[0]
            fid = mrow[1]
            nch = (ln + 127) // 128

            def chunk(ci, carry):
                pltpu.sync_copy(ssrc3.at[wid].at[ci], ssrc_c)
                pltpu.sync_copy(sdst3.at[wid].at[ci], sdst_c)
                pltpu.sync_copy(table.at[ssrc_c], rows_v)
                if scaled:
                    for g in range(8):
                        isrc = ssrc_c[pl.ds(g * 16, 16)]
                        idst = sdst_c[pl.ds(g * 16, 16)]
                        a = plsc.load_gather(dinv_v, [isrc])
                        b = plsc.load_gather(dinv_v, [idst])
                        norm_v[pl.ds(g * 16, 16)] = a * b
                ng = jnp.minimum(8, (ln - ci * 128) // 16)

                def grp(gi, car):
                    ddv = sdst_c[pl.ds(gi * 16, 16)]
                    if scaled:
                        nvv = norm_v[pl.ds(gi * 16, 16)]
                    for k in range(16):
                        prev = car[0]
                        dd = ddv[k]
                        off = jnp.clip(dd - fid, 0, _LOCAL - 1)
                        same = jnp.where(dd == prev, 1.0, 0.0)
                        ps = []
                        for g in range(W // 16):
                            r = rows_v[gi * 16 + k, pl.ds(g * 16, 16)]
                            if scaled:
                                r = r * nvv[k]
                            p = car[1 + g] * same + r
                            acc_v[off, pl.ds(g * 16, 16)] = p
                            ps.append(p)
                        car = (dd, *ps)
                    return car

                return lax.fori_loop(0, ng, grp, carry)

            z16 = jnp.zeros((16,), jnp.float32)
            init = (jnp.int32(-1),) + (z16,) * (W // 16)
            lax.fori_loop(0, nch, chunk, init)

            span = jnp.clip(mrow[2] - fid, 0, _LOCAL - 1)
            for g in range(W // 16):
                side_v[0, pl.ds(g * 16, 16)] = acc_v[0, pl.ds(g * 16, 16)]
                acc_v[0, pl.ds(g * 16, 16)] = z16
            for g in range(W // 16):
                side_v[1, pl.ds(g * 16, 16)] = acc_v[span, pl.ds(g * 16, 16)]
                acc_v[span, pl.ds(g * 16, 16)] = z16
            pltpu.sync_copy(side_v.at[pl.ds(0, 2)], out_side.at[wid])
            iota16 = lax.iota(jnp.int32, 16)
            for cc in range(_LOCAL // 128):
                for g in range(8):
                    idx_v[cc, pl.ds(g * 16, 16)] = fid + (cc * 128 + g * 16) + iota16
            for cc in range(_LOCAL // 128):
                pltpu.sync_copy(acc_v.at[pl.ds(cc * 128, 128)],
                                acc_sh.at[idx_v.at[cc]], add=True)

        plsc.subcore_barrier()
        nrt = 624
        pltpu.sync_copy(acc_sh.at[pl.ds(sid * nrt, nrt)],
                        out_acc.at[cid].at[pl.ds(sid * nrt, nrt)])

        @pl.when(sid == 0)
        def _tail():
            pltpu.sync_copy(acc_sh.at[pl.ds(16 * nrt, N - 16 * nrt)],
                            out_acc.at[cid].at[pl.ds(16 * nrt, N - 16 * nrt)])

    return body


_scat_deg = _make_sorted_scatter(16, 16, _SH_G, False)
_scat_gcn = _make_sorted_scatter(H, 16, _SH_G, True)
_scat_gru = _make_sorted_scatter(H, 32, _SH_E, False)


def _merge_sides(m_ref, side_ref, ids_ref, nsh, w):
    def mbody(k, carry):
        s = k // 2
        t = k - 2 * s
        idx = ids_ref[s, t]
        row = m_ref[pl.ds(idx, 1), :]
        m_ref[pl.ds(idx, 1), :] = row + side_ref[s, t, :].reshape(1, w)
        return carry

    lax.fori_loop(0, 2 * nsh, mbody, 0)


def _ka_body(dacc_ref, dside_ref, dids_ref, x_ref, w1_ref,
             xw_ref, dinv_ref, m_ref):
    m_ref[...] = dacc_ref[0] + dacc_ref[1]
    _merge_sides(m_ref, dside_ref, dids_ref, 16, 16)
    deg = m_ref[:, 0:1]
    dinv_ref[...] = lax.rsqrt(deg)
    xw_ref[...] = jnp.dot(x_ref[...], w1_ref[...],
                          preferred_element_type=jnp.float32)


def _kb_body(gacc_ref, gside_ref, gids_ref, b1_ref, wg0_ref,
             h_ref, tab_ref, m_ref):
    m_ref[...] = gacc_ref[0] + gacc_ref[1]
    _merge_sides(m_ref, gside_ref, gids_ref, 16, H)
    h0 = jnp.maximum(m_ref[...] + b1_ref[...], 0.0)
    h_ref[...] = h0
    tab_ref[...] = jnp.dot(h0, wg0_ref[...], preferred_element_type=jnp.float32)


def _kc_body(macc_ref, side_ref, ids_ref, h_ref, wih_ref, whh_ref,
             bih_ref, bhh_ref, wgn_ref, hout_ref, tab_ref, m_ref):
    m_ref[...] = macc_ref[0] + macc_ref[1]
    _merge_sides(m_ref, side_ref, ids_ref, 32, H)
    m = m_ref[...]
    h = h_ref[...]
    gi = jnp.dot(m, wih_ref[...], preferred_element_type=jnp.float32) + bih_ref[...]
    gh = jnp.dot(h, whh_ref[...], preferred_element_type=jnp.float32) + bhh_ref[...]
    r = jax.nn.sigmoid(gi[:, 0:H] + gh[:, 0:H])
    z = jax.nn.sigmoid(gi[:, H:2 * H] + gh[:, H:2 * H])
    n = jnp.tanh(gi[:, 2 * H:3 * H] + r * gh[:, 2 * H:3 * H])
    hn = (1.0 - z) * n + z * h
    hout_ref[...] = hn
    tab_ref[...] = jnp.dot(hn, wgn_ref[...], preferred_element_type=jnp.float32)


def _kd_body(h_ref, wf_ref, bf_ref, out_ref):
    hr = jnp.maximum(h_ref[...], 0.0)
    out_ref[...] = jnp.dot(hr, wf_ref[...], preferred_element_type=jnp.float32) + bf_ref[...]


def _smem_spec():
    return pl.BlockSpec(memory_space=pltpu.SMEM)


def _vmem_spec():
    return pl.BlockSpec(memory_space=pltpu.ANY)


_f32 = jnp.float32
_KA = pl.pallas_call(
    _ka_body,
    out_shape=[jax.ShapeDtypeStruct((N, H), _f32),
               jax.ShapeDtypeStruct((N, 1), _f32)],
    in_specs=[pl.BlockSpec(None), pl.BlockSpec(None), _smem_spec(),
              pl.BlockSpec(None), pl.BlockSpec(None)],
    scratch_shapes=[pltpu.VMEM((N, 16), _f32)],
)
_KB = pl.pallas_call(
    _kb_body,
    out_shape=[jax.ShapeDtypeStruct((N, H), _f32)] * 2,
    in_specs=[pl.BlockSpec(None), pl.BlockSpec(None), _smem_spec(),
              pl.BlockSpec(None), pl.BlockSpec(None)],
    scratch_shapes=[pltpu.VMEM((N, H), _f32)],
)
_KC = pl.pallas_call(
    _kc_body,
    out_shape=[jax.ShapeDtypeStruct((N, H), _f32)] * 2,
    in_specs=[pl.BlockSpec(None), pl.BlockSpec(None), _smem_spec()]
    + [pl.BlockSpec(None)] * 6,
    scratch_shapes=[pltpu.VMEM((N, H), _f32)],
)
_KD = pl.pallas_call(
    _kd_body,
    out_shape=jax.ShapeDtypeStruct((N, C), _f32),
)


def _pack(arr, cuts, shmax):
    segs = []
    for i in range(len(cuts) - 1):
        a, b = cuts[i], cuts[i + 1]
        segs.append(jnp.pad(arr[a:b], (0, shmax * 128 - (b - a))))
    return jnp.stack(segs).reshape(len(cuts) - 1, shmax, 128)


def _meta(sidx, cuts):
    lens = jnp.array([cuts[i + 1] - cuts[i] for i in range(len(cuts) - 1)],
                     jnp.int32)
    starts = jnp.array(cuts[:-1], jnp.int32)
    ends = jnp.array(cuts[1:], jnp.int32) - 1
    firsts = sidx[starts]
    lasts = sidx[ends]
    pad = [jnp.zeros_like(lens)] * 13
    meta = jnp.stack([lens, firsts, lasts] + pad, axis=1)
    ids = jnp.stack([firsts, lasts], axis=1)
    return meta, ids


def kernel(x, edge_index, batch, W1, b1, Wg, Wih, Whh, bih, bhh, Wf, bf):
    src = edge_index[0]
    dst = edge_index[1]
    iN = jnp.arange(N, dtype=jnp.int32)

    perm = jnp.argsort(dst, stable=True)
    ssrc = src[perm]
    sdst = dst[perm]
    srcf = jnp.concatenate([src, iN])
    dstf = jnp.concatenate([dst, iN])
    permf = jnp.argsort(dstf, stable=True)
    ssrcf = srcf[permf]
    sdstf = dstf[permf]

    ssrc3 = _pack(ssrc, _CUTS_E, _SH_E)
    sdst3 = _pack(sdst, _CUTS_E, _SH_E)
    ssrcf3 = _pack(ssrcf, _CUTS_G, _SH_G)
    sdstf3 = _pack(sdstf, _CUTS_G, _SH_G)
    meta_e, ids_e = _meta(sdst, _CUTS_E)
    meta_g, ids_g = _meta(sdstf, _CUTS_G)

    zero16 = jnp.zeros((_LOCAL, 16), _f32)
    zero32 = jnp.zeros((_LOCAL, H), _f32)
    ones16 = jnp.ones((N, 16), _f32)

    b1_2 = b1.reshape(1, H)
    bih_2 = bih.reshape(1, 3 * H)
    bhh_2 = bhh.reshape(1, 3 * H)
    bf_2 = bf.reshape(1, C)
    wih_t = Wih.T
    whh_t = Whh.T

    dacc, dside = _scat_deg(ones16, ssrcf3, sdstf3, meta_g, zero16)
    xw, dinv = _KA(dacc, dside, ids_g, x, W1)
    gacc, gside = _scat_gcn(xw, ssrcf3, sdstf3, meta_g, zero32,
                            dinv.reshape(N))
    h, tab = _KB(gacc, gside, ids_g, b1_2, Wg[0])
    for i in range(L):
        macc, mside = _scat_gru(tab, ssrc3, sdst3, meta_e, zero32)
        h, tab = _KC(macc, mside, ids_e, h, wih_t, whh_t, bih_2, bhh_2,
                     Wg[(i + 1) % L])
    return _KD(h, Wf, bf_2)


# speedup vs baseline: 7.7567x; 1.7299x over previous
import functools

import jax
import jax.numpy as jnp
from jax import lax
from jax.experimental import pallas as pl
from jax.experimental.pallas import tpu as pltpu
from jax.experimental.pallas import tpu_sc as plsc

N = 10000
E = 320000
D = 128
H = 32
C = 7
L = 32

_NPAD = 12288
_ZR = _NPAD // 16
_LOCAL = 2048

_CUTS_E = [0]
for _sc in (0, 160000):
    _CUTS_E += [_sc + 10368 + 9984 * k for k in range(15)] + [_sc + 160000]
_CUTS_G = [0] + [20736 * k for k in range(1, 13)] \
    + [248832 + 20352 * k for k in range(1, 4)] + [330000]

_SH_E = 81
_SH_G = 162


def _make_sorted_scatter(W, nsh, shmax, scaled):
    mesh = plsc.VectorSubcoreMesh(core_axis_name="c", subcore_axis_name="s")

    out_type = [
        jax.ShapeDtypeStruct((2, N, W), jnp.float32),
        jax.ShapeDtypeStruct((nsh, 2, W), jnp.float32),
    ]
    scratch = [
        pltpu.VMEM((3, 128), jnp.int32),
        pltpu.VMEM((3, 128), jnp.int32),
        pltpu.VMEM((2, 128, W), jnp.float32),
        pltpu.VMEM((_LOCAL, W), jnp.float32),
        pltpu.VMEM((8, W), jnp.float32),
        pltpu.VMEM((nsh, 16), jnp.int32),
        pltpu.VMEM((_LOCAL // 128, 128), jnp.int32),
        pltpu.VMEM_SHARED((_NPAD, W), jnp.float32),
        pltpu.SemaphoreType.DMA,
        pltpu.SemaphoreType.DMA,
    ]
    if scaled:
        scratch += [pltpu.VMEM((N,), jnp.float32),
                    pltpu.VMEM((128,), jnp.float32)]

    @functools.partial(
        pl.kernel,
        out_type=out_type,
        mesh=mesh,
        scratch_types=scratch,
        compiler_params=pltpu.CompilerParams(use_tc_tiling_on_sc=False, needs_layout_passes=False),
    )
    def body(table, ssrc3, sdst3, meta, zero, *rest):
        if scaled:
            (dinv, out_acc, out_side, ssrc_c, sdst_c, rows_v, acc_v, side_v,
             meta_v, idx_v, acc_sh, sem_i, sem_r, dinv_v, norm_v) = rest
        else:
            (out_acc, out_side, ssrc_c, sdst_c, rows_v, acc_v, side_v,
             meta_v, idx_v, acc_sh, sem_i, sem_r) = rest
        cid = lax.axis_index("c")
        sid = lax.axis_index("s")
        wid = sid * 2 + cid
        pltpu.sync_copy(zero.at[pl.ds(0, _ZR)], acc_sh.at[pl.ds(sid * _ZR, _ZR)])

        @pl.when(wid < nsh)
        def _work():
            pltpu.sync_copy(zero, acc_v)
            pltpu.sync_copy(meta, meta_v)
            if scaled:
                pltpu.sync_copy(dinv, dinv_v)
            mrow = meta_v[wid]
            ln = mrow[0]
            fid = mrow[1]
            nch = (ln + 127) // 128

            def start_idx(ci):
                b = lax.rem(ci, 3)
                pltpu.async_copy(ssrc3.at[wid].at[ci], ssrc_c.at[b], sem_i)
                pltpu.async_copy(sdst3.at[wid].at[ci], sdst_c.at[b], sem_i)

            def wait_idx(ci):
                b = lax.rem(ci, 3)
                pltpu.make_async_copy(ssrc3.at[wid].at[ci], ssrc_c.at[b],
                                      sem_i).wait()
                pltpu.make_async_copy(sdst3.at[wid].at[ci], sdst_c.at[b],
                                      sem_i).wait()

            def start_rows(ci):
                b3 = lax.rem(ci, 3)
                b2 = lax.rem(ci, 2)
                pltpu.async_copy(table.at[ssrc_c.at[b3]], rows_v.at[b2], sem_r)

            def wait_rows(ci):
                b3 = lax.rem(ci, 3)
                b2 = lax.rem(ci, 2)
                pltpu.make_async_copy(table.at[ssrc_c.at[b3]], rows_v.at[b2],
                                      sem_r).wait()

            start_idx(jnp.int32(0))

            @pl.when(nch > 1)
            def _pr1():
                start_idx(jnp.int32(1))

            wait_idx(jnp.int32(0))
            start_rows(jnp.int32(0))

            def chunk(ci, carry):
                b3 = lax.rem(ci, 3)
                b2 = lax.rem(ci, 2)
                wait_rows(ci)

                @pl.when(ci + 1 < nch)
                def _nx():
                    wait_idx(ci + 1)
                    start_rows(ci + 1)

                @pl.when(ci + 2 < nch)
                def _pf():
                    start_idx(ci + 2)

                if scaled:
                    for g in range(8):
                        isrc = ssrc_c[b3, pl.ds(g * 16, 16)]
                        idst = sdst_c[b3, pl.ds(g * 16, 16)]
                        a = plsc.load_gather(dinv_v, [isrc])
                        b = plsc.load_gather(dinv_v, [idst])
                        norm_v[pl.ds(g * 16, 16)] = a * b
                ng = jnp.minimum(8, (ln - ci * 128) // 16)

                def grp(gi, car):
                    ddv = sdst_c[b3, pl.ds(gi * 16, 16)]
                    if scaled:
                        nvv = norm_v[pl.ds(gi * 16, 16)]
                    for k in range(16):
                        prev = car[0]
                        dd = ddv[k]
                        off = jnp.clip(dd - fid, 0, _LOCAL - 1)
                        same = jnp.where(dd == prev, 1.0, 0.0)
                        ps = []
                        for g in range(W // 16):
                            r = rows_v[b2, gi * 16 + k, pl.ds(g * 16, 16)]
                            if scaled:
                                r = r * nvv[k]
                            p = car[1 + g] * same + r
                            acc_v[off, pl.ds(g * 16, 16)] = p
                            ps.append(p)
                        car = (dd, *ps)
                    return car

                return lax.fori_loop(0, ng, grp, carry)

            z16 = jnp.zeros((16,), jnp.float32)
            init = (jnp.int32(-1),) + (z16,) * (W // 16)
            lax.fori_loop(0, nch, chunk, init)

            span = jnp.clip(mrow[2] - fid, 0, _LOCAL - 1)
            for g in range(W // 16):
                side_v[0, pl.ds(g * 16, 16)] = acc_v[0, pl.ds(g * 16, 16)]
                acc_v[0, pl.ds(g * 16, 16)] = z16
            for g in range(W // 16):
                side_v[1, pl.ds(g * 16, 16)] = acc_v[span, pl.ds(g * 16, 16)]
                acc_v[span, pl.ds(g * 16, 16)] = z16
            pltpu.sync_copy(side_v.at[pl.ds(0, 2)], out_side.at[wid])
            iota16 = lax.iota(jnp.int32, 16)
            for cc in range(_LOCAL // 128):
                for g in range(8):
                    idx_v[cc, pl.ds(g * 16, 16)] = fid + (cc * 128 + g * 16) + iota16
            for cc in range(_LOCAL // 128):
                pltpu.sync_copy(acc_v.at[pl.ds(cc * 128, 128)],
                                acc_sh.at[idx_v.at[cc]], add=True)

        plsc.subcore_barrier()
        nrt = 624
        pltpu.sync_copy(acc_sh.at[pl.ds(sid * nrt, nrt)],
                        out_acc.at[cid].at[pl.ds(sid * nrt, nrt)])

        @pl.when(sid == 0)
        def _tail():
            pltpu.sync_copy(acc_sh.at[pl.ds(16 * nrt, N - 16 * nrt)],
                            out_acc.at[cid].at[pl.ds(16 * nrt, N - 16 * nrt)])

    return body


_scat_deg = _make_sorted_scatter(16, 16, _SH_G, False)
_scat_gcn = _make_sorted_scatter(H, 16, _SH_G, True)
_scat_gru = _make_sorted_scatter(H, 32, _SH_E, False)


def _merge_sides(m_ref, side_ref, ids_ref, nsh, w):
    def mbody(k, carry):
        s = k // 2
        t = k - 2 * s
        idx = ids_ref[s, t]
        row = m_ref[pl.ds(idx, 1), :]
        m_ref[pl.ds(idx, 1), :] = row + side_ref[s, t, :].reshape(1, w)
        return carry

    lax.fori_loop(0, 2 * nsh, mbody, 0)


def _ka_body(dacc_ref, dside_ref, dids_ref, x_ref, w1_ref,
             xw_ref, dinv_ref, m_ref):
    m_ref[...] = dacc_ref[0] + dacc_ref[1]
    _merge_sides(m_ref, dside_ref, dids_ref, 16, 16)
    deg = m_ref[:, 0:1]
    dinv_ref[...] = lax.rsqrt(deg)
    xw_ref[...] = jnp.dot(x_ref[...], w1_ref[...],
                          preferred_element_type=jnp.float32)


def _kb_body(gacc_ref, gside_ref, gids_ref, b1_ref, wg0_ref,
             h_ref, tab_ref, m_ref):
    m_ref[...] = gacc_ref[0] + gacc_ref[1]
    _merge_sides(m_ref, gside_ref, gids_ref, 16, H)
    h0 = jnp.maximum(m_ref[...] + b1_ref[...], 0.0)
    h_ref[...] = h0
    tab_ref[...] = jnp.dot(h0, wg0_ref[...], preferred_element_type=jnp.float32)


def _kc_body(macc_ref, side_ref, ids_ref, h_ref, wih_ref, whh_ref,
             bih_ref, bhh_ref, wgn_ref, hout_ref, tab_ref, m_ref):
    m_ref[...] = macc_ref[0] + macc_ref[1]
    _merge_sides(m_ref, side_ref, ids_ref, 32, H)
    m = m_ref[...]
    h = h_ref[...]
    gi = jnp.dot(m, wih_ref[...], preferred_element_type=jnp.float32) + bih_ref[...]
    gh = jnp.dot(h, whh_ref[...], preferred_element_type=jnp.float32) + bhh_ref[...]
    r = jax.nn.sigmoid(gi[:, 0:H] + gh[:, 0:H])
    z = jax.nn.sigmoid(gi[:, H:2 * H] + gh[:, H:2 * H])
    n = jnp.tanh(gi[:, 2 * H:3 * H] + r * gh[:, 2 * H:3 * H])
    hn = (1.0 - z) * n + z * h
    hout_ref[...] = hn
    tab_ref[...] = jnp.dot(hn, wgn_ref[...], preferred_element_type=jnp.float32)


def _kd_body(h_ref, wf_ref, bf_ref, out_ref):
    hr = jnp.maximum(h_ref[...], 0.0)
    out_ref[...] = jnp.dot(hr, wf_ref[...], preferred_element_type=jnp.float32) + bf_ref[...]


def _smem_spec():
    return pl.BlockSpec(memory_space=pltpu.SMEM)


def _vmem_spec():
    return pl.BlockSpec(memory_space=pltpu.ANY)


_f32 = jnp.float32
_KA = pl.pallas_call(
    _ka_body,
    out_shape=[jax.ShapeDtypeStruct((N, H), _f32),
               jax.ShapeDtypeStruct((N, 1), _f32)],
    in_specs=[pl.BlockSpec(None), pl.BlockSpec(None), _smem_spec(),
              pl.BlockSpec(None), pl.BlockSpec(None)],
    scratch_shapes=[pltpu.VMEM((N, 16), _f32)],
)
_KB = pl.pallas_call(
    _kb_body,
    out_shape=[jax.ShapeDtypeStruct((N, H), _f32)] * 2,
    in_specs=[pl.BlockSpec(None), pl.BlockSpec(None), _smem_spec(),
              pl.BlockSpec(None), pl.BlockSpec(None)],
    scratch_shapes=[pltpu.VMEM((N, H), _f32)],
)
_KC = pl.pallas_call(
    _kc_body,
    out_shape=[jax.ShapeDtypeStruct((N, H), _f32)] * 2,
    in_specs=[pl.BlockSpec(None), pl.BlockSpec(None), _smem_spec()]
    + [pl.BlockSpec(None)] * 6,
    scratch_shapes=[pltpu.VMEM((N, H), _f32)],
)
_KD = pl.pallas_call(
    _kd_body,
    out_shape=jax.ShapeDtypeStruct((N, C), _f32),
)


def _pack(arr, cuts, shmax):
    segs = []
    for i in range(len(cuts) - 1):
        a, b = cuts[i], cuts[i + 1]
        segs.append(jnp.pad(arr[a:b], (0, shmax * 128 - (b - a))))
    return jnp.stack(segs).reshape(len(cuts) - 1, shmax, 128)


def _meta(sidx, cuts):
    lens = jnp.array([cuts[i + 1] - cuts[i] for i in range(len(cuts) - 1)],
                     jnp.int32)
    starts = jnp.array(cuts[:-1], jnp.int32)
    ends = jnp.array(cuts[1:], jnp.int32) - 1
    firsts = sidx[starts]
    lasts = sidx[ends]
    pad = [jnp.zeros_like(lens)] * 13
    meta = jnp.stack([lens, firsts, lasts] + pad, axis=1)
    ids = jnp.stack([firsts, lasts], axis=1)
    return meta, ids


def kernel(x, edge_index, batch, W1, b1, Wg, Wih, Whh, bih, bhh, Wf, bf):
    src = edge_index[0]
    dst = edge_index[1]
    iN = jnp.arange(N, dtype=jnp.int32)

    perm = jnp.argsort(dst, stable=True)
    ssrc = src[perm]
    sdst = dst[perm]
    srcf = jnp.concatenate([src, iN])
    dstf = jnp.concatenate([dst, iN])
    permf = jnp.argsort(dstf, stable=True)
    ssrcf = srcf[permf]
    sdstf = dstf[permf]

    ssrc3 = _pack(ssrc, _CUTS_E, _SH_E)
    sdst3 = _pack(sdst, _CUTS_E, _SH_E)
    ssrcf3 = _pack(ssrcf, _CUTS_G, _SH_G)
    sdstf3 = _pack(sdstf, _CUTS_G, _SH_G)
    meta_e, ids_e = _meta(sdst, _CUTS_E)
    meta_g, ids_g = _meta(sdstf, _CUTS_G)

    zero16 = jnp.zeros((_LOCAL, 16), _f32)
    zero32 = jnp.zeros((_LOCAL, H), _f32)
    ones16 = jnp.ones((N, 16), _f32)

    b1_2 = b1.reshape(1, H)
    bih_2 = bih.reshape(1, 3 * H)
    bhh_2 = bhh.reshape(1, 3 * H)
    bf_2 = bf.reshape(1, C)
    wih_t = Wih.T
    whh_t = Whh.T

    dacc, dside = _scat_deg(ones16, ssrcf3, sdstf3, meta_g, zero16)
    xw, dinv = _KA(dacc, dside, ids_g, x, W1)
    gacc, gside = _scat_gcn(xw, ssrcf3, sdstf3, meta_g, zero32,
                            dinv.reshape(N))
    h, tab = _KB(gacc, gside, ids_g, b1_2, Wg[0])
    for i in range(L):
        macc, mside = _scat_gru(tab, ssrc3, sdst3, meta_e, zero32)
        h, tab = _KC(macc, mside, ids_e, h, wih_t, whh_t, bih_2, bhh_2,
                     Wg[(i + 1) % L])
    return _KD(h, Wf, bf_2)
